# baseline (speedup 1.0000x reference)
import functools

import jax
import jax.numpy as jnp
from jax import lax
from jax.experimental import pallas as pl
from jax.experimental.pallas import tpu as pltpu
from jax.experimental.pallas import tpu_sc as plsc

N = 10000
E = 160000
D_IN = 256
D_H = 256
D_OUT = 64

NS = 16
NC = 2
NB = 5

KD = 80
CCD = (E // NS) // KD

KH = 80
CCH = (E // NS) // KH

DQ2 = D_OUT // 4

RCH = 624
RTAIL = N - NS * RCH

BM = 400
GRID = N // BM

_mesh = plsc.VectorSubcoreMesh(core_axis_name="c", subcore_axis_name="s")
_params = pltpu.CompilerParams(use_tc_tiling_on_sc=False)


def _zero_acc(zrows, acc, s):
    pltpu.sync_copy(zrows.at[pl.ds(0, RCH)], acc.at[pl.ds(s * RCH, RCH)])

    @pl.when(s == NS - 1)
    def _():
        pltpu.sync_copy(zrows.at[pl.ds(0, RTAIL)],
                        acc.at[pl.ds(NS * RCH, RTAIL)])


def _dump_acc(acc, out, s):
    pltpu.sync_copy(acc.at[pl.ds(s * RCH, RCH)], out.at[pl.ds(s * RCH, RCH)])

    @pl.when(s == NS - 1)
    def _():
        pltpu.sync_copy(acc.at[pl.ds(NS * RCH, RTAIL)],
                        out.at[pl.ds(NS * RCH, RTAIL)])


def _edge_loop(hw, acc, idx_s, idx_d, gbufs, gsems, ssems, cch):
    nb = len(gbufs)
    t_outer = cch // nb
    for b in range(nb - 1):
        pltpu.async_copy(hw.at[idx_s.at[b]], gbufs[b], gsems[b])

    def outer(t, carry):
        for b in range(nb):
            jj = t * nb + b
            bprev = (b - 1) % nb

            def wait_scatter(bp=bprev, j=jj):
                pltpu.make_async_copy(gbufs[bp], acc.at[idx_d.at[j - 1]],
                                      ssems[bp]).wait()

            def issue_gather(bp=bprev, j=jj):
                pltpu.async_copy(hw.at[idx_s.at[j + nb - 1]], gbufs[bp],
                                 gsems[bp])

            if b == 0:
                pl.when(t > 0)(wait_scatter)
                issue_gather()
            else:
                wait_scatter()
                pl.when(t < t_outer - 1)(issue_gather)
            pltpu.make_async_copy(hw.at[idx_s.at[jj]], gbufs[b],
                                  gsems[b]).wait()
            pltpu.async_copy(gbufs[b], acc.at[idx_d.at[jj]], ssems[b],
                             add=True)
        return carry

    lax.fori_loop(0, t_outer, outer, 0)
    bl = (cch - 1) % nb
    pltpu.make_async_copy(gbufs[bl], acc.at[idx_d.at[cch - 1]],
                          ssems[bl]).wait()


@functools.partial(
    pl.kernel,
    out_type=jax.ShapeDtypeStruct((NC, N, 16), jnp.float32),
    mesh=_mesh,
    compiler_params=_params,
    scratch_types=[
        pltpu.VMEM((CCD, KD), jnp.int32),
        pltpu.VMEM((KD, 16), jnp.float32),
        pltpu.VMEM_SHARED((N, 16), jnp.float32),
        [pltpu.SemaphoreType.DMA] * NB,
    ],
)
def _deg_kernel(e4, zrows, ones_rows, deg_out, idx, ones_v, deg_sp, dsems):
    c = lax.axis_index("c")
    s = lax.axis_index("s")
    _zero_acc(zrows, deg_sp, s)
    pltpu.sync_copy(e4.at[c, s], idx)
    pltpu.sync_copy(ones_rows, ones_v)
    plsc.subcore_barrier()

    def outer(t, carry):
        for b in range(NB):
            jj = t * NB + b

            def wait_prev(b=b, jj=jj):
                pltpu.make_async_copy(ones_v, deg_sp.at[idx.at[jj - NB]],
                                      dsems[b]).wait()

            pl.when(t > 0)(wait_prev)
            pltpu.async_copy(ones_v, deg_sp.at[idx.at[jj]], dsems[b],
                             add=True)
        return carry

    lax.fori_loop(0, CCD // NB, outer, 0)
    for b in range(NB):
        pltpu.make_async_copy(ones_v, deg_sp.at[idx.at[CCD - NB + b]],
                              dsems[b]).wait()
    plsc.subcore_barrier()
    _dump_acc(deg_sp, deg_out.at[c], s)


@functools.partial(
    pl.kernel,
    out_type=[jax.ShapeDtypeStruct((N, D_H // 4), jnp.float32)] * 4,
    mesh=_mesh,
    compiler_params=_params,
    scratch_types=[
        pltpu.VMEM((CCH, KH), jnp.int32),
        pltpu.VMEM((CCH, KH), jnp.int32),
        [pltpu.VMEM((KH, D_H // 4), jnp.float32)] * NB,
        pltpu.VMEM_SHARED((N, D_H // 4), jnp.float32),
        [pltpu.SemaphoreType.DMA] * NB,
        [pltpu.SemaphoreType.DMA] * NB,
    ],
)
def _agg_h(hw0, hw1, hw2, hw3, srcH, dstH, zrows,
           out0, out1, out2, out3, idx_s, idx_d, gbufs, acc, gsems, ssems):
    c = lax.axis_index("c")
    s = lax.axis_index("s")
    pltpu.sync_copy(srcH.at[s], idx_s)
    pltpu.sync_copy(dstH.at[s], idx_d)

    def run(hw, out):
        _zero_acc(zrows, acc, s)
        plsc.subcore_barrier()
        _edge_loop(hw, acc, idx_s, idx_d, gbufs, gsems, ssems, CCH)
        plsc.subcore_barrier()
        _dump_acc(acc, out, s)
        plsc.subcore_barrier()

    @pl.when(c == 0)
    def _():
        run(hw0, out0)
        run(hw1, out1)

    @pl.when(c == 1)
    def _():
        run(hw2, out2)
        run(hw3, out3)


@functools.partial(
    pl.kernel,
    out_type=jax.ShapeDtypeStruct((N, D_OUT), jnp.float32),
    mesh=_mesh,
    compiler_params=_params,
    scratch_types=[
        pltpu.VMEM((CCH, KH), jnp.int32),
        pltpu.VMEM((CCH, KH), jnp.int32),
        [pltpu.VMEM((KH, DQ2), jnp.float32)] * NB,
        pltpu.VMEM_SHARED((N, DQ2), jnp.float32),
        pltpu.VMEM((RCH + RTAIL, DQ2), jnp.float32),
        pltpu.VMEM((RCH + RTAIL, DQ2), jnp.float32),
        pltpu.VMEM((4, DQ2), jnp.float32),
        [pltpu.SemaphoreType.DMA] * NB,
        [pltpu.SemaphoreType.DMA] * NB,
    ],
)
def _agg_o(g0, g1, g2, g3, srcH, dstH, zrows, nrm16, b2q, out,
           idx_s, idx_d, gbufs, acc, tbuf, nbuf, b2v, gsems, ssems):
    c = lax.axis_index("c")
    s = lax.axis_index("s")
    pltpu.sync_copy(srcH.at[s], idx_s)
    pltpu.sync_copy(dstH.at[s], idx_d)
    pltpu.sync_copy(b2q, b2v)
    pltpu.sync_copy(nrm16.at[pl.ds(s * RCH, RCH)], nbuf.at[pl.ds(0, RCH)])

    @pl.when(s == NS - 1)
    def _():
        pltpu.sync_copy(nrm16.at[pl.ds(NS * RCH, RTAIL)],
                        nbuf.at[pl.ds(RCH, RTAIL)])

    def scale_rows(lo, nrows, b2row):
        def body(r, carry):
            tbuf[r] = tbuf[r] * nbuf[r] + b2row
            return carry
        lax.fori_loop(lo, lo + nrows, body, 0)

    def run(g, q):
        _zero_acc(zrows, acc, s)
        plsc.subcore_barrier()
        _edge_loop(g, acc, idx_s, idx_d, gbufs, gsems, ssems, CCH)
        plsc.subcore_barrier()
        pltpu.sync_copy(acc.at[pl.ds(s * RCH, RCH)], tbuf.at[pl.ds(0, RCH)])

        @pl.when(s == NS - 1)
        def _():
            pltpu.sync_copy(acc.at[pl.ds(NS * RCH, RTAIL)],
                            tbuf.at[pl.ds(RCH, RTAIL)])

        plsc.subcore_barrier()
        b2row = b2v[q]
        scale_rows(0, RCH, b2row)

        @pl.when(s == NS - 1)
        def _():
            scale_rows(RCH, RTAIL, b2row)

        pltpu.sync_copy(tbuf.at[pl.ds(0, RCH)],
                        out.at[pl.ds(s * RCH, RCH), pl.ds(q * DQ2, DQ2)])

        @pl.when(s == NS - 1)
        def _():
            pltpu.sync_copy(tbuf.at[pl.ds(RCH, RTAIL)],
                            out.at[pl.ds(NS * RCH, RTAIL),
                                   pl.ds(q * DQ2, DQ2)])

    @pl.when(c == 0)
    def _():
        run(g0, 0)
        run(g1, 1)

    @pl.when(c == 1)
    def _():
        run(g2, 2)
        run(g3, 3)


def _tc1_body(x_ref, deg_ref, w_ref, o0, o1, o2, o3):
    n_out = lax.rsqrt(jnp.maximum(deg_ref[0, :, 0:1], 1.0))
    y = jnp.dot(x_ref[...] * n_out, w_ref[...],
                preferred_element_type=jnp.float32)
    dq = D_H // 4
    for q, o in enumerate((o0, o1, o2, o3)):
        o[...] = y[:, q * dq:(q + 1) * dq]


def _tc1(x, deg16, w1):
    return pl.pallas_call(
        _tc1_body,
        grid=(GRID,),
        in_specs=[
            pl.BlockSpec((BM, D_IN), lambda i: (i, 0)),
            pl.BlockSpec((1, BM, 16), lambda i: (0, i, 0)),
            pl.BlockSpec((D_IN, D_H), lambda i: (0, 0)),
        ],
        out_specs=[pl.BlockSpec((BM, D_H // 4), lambda i: (i, 0))] * 4,
        out_shape=[jax.ShapeDtypeStruct((N, D_H // 4), jnp.float32)] * 4,
    )(x, deg16, w1)


def _tc2_body(a0, a1, a2, a3, deg_ref, b1_ref, w_ref, o0, o1, o2, o3, on):
    n_out = lax.rsqrt(jnp.maximum(deg_ref[0, :, 0:1], 1.0))
    n_in = lax.rsqrt(jnp.maximum(deg_ref[1, :, 0:1], 1.0))
    agg = jnp.concatenate([a0[...], a1[...], a2[...], a3[...]], axis=1)
    h = jax.nn.relu(agg * n_in + b1_ref[...]) * n_out
    y = jnp.dot(h, w_ref[...], preferred_element_type=jnp.float32)
    for q, o in enumerate((o0, o1, o2, o3)):
        o[...] = y[:, q * DQ2:(q + 1) * DQ2]
    on[...] = jnp.broadcast_to(n_in, (BM, 16))


def _tc2(aggs, deg16, b1, w2):
    return pl.pallas_call(
        _tc2_body,
        grid=(GRID,),
        in_specs=[pl.BlockSpec((BM, D_H // 4), lambda i: (i, 0))] * 4 + [
            pl.BlockSpec((2, BM, 16), lambda i: (0, i, 0)),
            pl.BlockSpec((1, D_H), lambda i: (0, 0)),
            pl.BlockSpec((D_H, D_OUT), lambda i: (0, 0)),
        ],
        out_specs=[pl.BlockSpec((BM, DQ2), lambda i: (i, 0))] * 4 + [
            pl.BlockSpec((BM, 16), lambda i: (i, 0))],
        out_shape=[jax.ShapeDtypeStruct((N, DQ2), jnp.float32)] * 4 + [
            jax.ShapeDtypeStruct((N, 16), jnp.float32)],
    )(*aggs, deg16, b1, w2)


def kernel(in_feat, edge_index, W1, b1, W2, b2):
    e4d = edge_index.reshape(NC, NS, CCD, KD)
    srcH = edge_index[0].reshape(NS, CCH, KH)
    dstH = edge_index[1].reshape(NS, CCH, KH)
    z16 = jnp.zeros((RCH, 16), jnp.float32)
    zh = jnp.zeros((RCH, D_H // 4), jnp.float32)
    zo = jnp.zeros((RCH, DQ2), jnp.float32)
    ones_rows = jnp.zeros((KD, 16), jnp.float32).at[:, 0].set(1.0)

    deg16 = _deg_kernel(e4d, z16, ones_rows)
    hw = _tc1(in_feat, deg16, W1)
    a1 = _agg_h(*hw, srcH, dstH, zh)
    g0, g1, g2, g3, nrm16 = _tc2(a1, deg16, b1.reshape(1, D_H), W2)
    return _agg_o(g0, g1, g2, g3, srcH, dstH, zo, nrm16,
                  b2.reshape(4, DQ2))


# speedup vs baseline: 1.1134x; 1.1134x over previous
import functools

import jax
import jax.numpy as jnp
from jax import lax
from jax.experimental import pallas as pl
from jax.experimental.pallas import tpu as pltpu
from jax.experimental.pallas import tpu_sc as plsc

N = 10000
E = 160000
D_IN = 256
D_H = 256
D_OUT = 64

NS = 16
NC = 2
NB = 5

KD = 80
CCD = (E // NS) // KD

KH = 80
CCH = (E // NS) // KH

DH2 = D_OUT // 2

RCH = 624
RTAIL = N - NS * RCH

BM = 400
GRID = N // BM

_mesh = plsc.VectorSubcoreMesh(core_axis_name="c", subcore_axis_name="s")
_params = pltpu.CompilerParams(use_tc_tiling_on_sc=False)


def _zero_acc(zrows, acc, s):
    pltpu.sync_copy(zrows.at[pl.ds(0, RCH)], acc.at[pl.ds(s * RCH, RCH)])

    @pl.when(s == NS - 1)
    def _():
        pltpu.sync_copy(zrows.at[pl.ds(0, RTAIL)],
                        acc.at[pl.ds(NS * RCH, RTAIL)])


def _dump_acc(acc, out, s):
    pltpu.sync_copy(acc.at[pl.ds(s * RCH, RCH)], out.at[pl.ds(s * RCH, RCH)])

    @pl.when(s == NS - 1)
    def _():
        pltpu.sync_copy(acc.at[pl.ds(NS * RCH, RTAIL)],
                        out.at[pl.ds(NS * RCH, RTAIL)])


def _edge_loop(hw, acc, idx_s, idx_d, gbufs, gsems, ssems, cch):
    nb = len(gbufs)
    t_outer = cch // nb
    for b in range(nb - 1):
        pltpu.async_copy(hw.at[idx_s.at[b]], gbufs[b], gsems[b])

    def outer(t, carry):
        for b in range(nb):
            jj = t * nb + b
            bprev = (b - 1) % nb

            def wait_scatter(bp=bprev, j=jj):
                pltpu.make_async_copy(gbufs[bp], acc.at[idx_d.at[j - 1]],
                                      ssems[bp]).wait()

            def issue_gather(bp=bprev, j=jj):
                pltpu.async_copy(hw.at[idx_s.at[j + nb - 1]], gbufs[bp],
                                 gsems[bp])

            if b == 0:
                pl.when(t > 0)(wait_scatter)
                issue_gather()
            else:
                wait_scatter()
                pl.when(t < t_outer - 1)(issue_gather)
            pltpu.make_async_copy(hw.at[idx_s.at[jj]], gbufs[b],
                                  gsems[b]).wait()
            pltpu.async_copy(gbufs[b], acc.at[idx_d.at[jj]], ssems[b],
                             add=True)
        return carry

    lax.fori_loop(0, t_outer, outer, 0)
    bl = (cch - 1) % nb
    pltpu.make_async_copy(gbufs[bl], acc.at[idx_d.at[cch - 1]],
                          ssems[bl]).wait()


@functools.partial(
    pl.kernel,
    out_type=jax.ShapeDtypeStruct((NC, N, 16), jnp.float32),
    mesh=_mesh,
    compiler_params=_params,
    scratch_types=[
        pltpu.VMEM((CCD, KD), jnp.int32),
        pltpu.VMEM((KD, 16), jnp.float32),
        pltpu.VMEM_SHARED((N, 16), jnp.float32),
        [pltpu.SemaphoreType.DMA] * NB,
    ],
)
def _deg_kernel(e4, zrows, ones_rows, deg_out, idx, ones_v, deg_sp, dsems):
    c = lax.axis_index("c")
    s = lax.axis_index("s")
    _zero_acc(zrows, deg_sp, s)
    pltpu.sync_copy(e4.at[c, s], idx)
    pltpu.sync_copy(ones_rows, ones_v)
    plsc.subcore_barrier()

    def outer(t, carry):
        for b in range(NB):
            jj = t * NB + b

            def wait_prev(b=b, jj=jj):
                pltpu.make_async_copy(ones_v, deg_sp.at[idx.at[jj - NB]],
                                      dsems[b]).wait()

            pl.when(t > 0)(wait_prev)
            pltpu.async_copy(ones_v, deg_sp.at[idx.at[jj]], dsems[b],
                             add=True)
        return carry

    lax.fori_loop(0, CCD // NB, outer, 0)
    for b in range(NB):
        pltpu.make_async_copy(ones_v, deg_sp.at[idx.at[CCD - NB + b]],
                              dsems[b]).wait()
    plsc.subcore_barrier()
    _dump_acc(deg_sp, deg_out.at[c], s)


@functools.partial(
    pl.kernel,
    out_type=[jax.ShapeDtypeStruct((N, D_H // 4), jnp.float32)] * 4,
    mesh=_mesh,
    compiler_params=_params,
    scratch_types=[
        pltpu.VMEM((CCH, KH), jnp.int32),
        pltpu.VMEM((CCH, KH), jnp.int32),
        [pltpu.VMEM((KH, D_H // 4), jnp.float32)] * NB,
        pltpu.VMEM_SHARED((N, D_H // 4), jnp.float32),
        [pltpu.SemaphoreType.DMA] * NB,
        [pltpu.SemaphoreType.DMA] * NB,
    ],
)
def _agg_h(hw0, hw1, hw2, hw3, srcH, dstH, zrows,
           out0, out1, out2, out3, idx_s, idx_d, gbufs, acc, gsems, ssems):
    c = lax.axis_index("c")
    s = lax.axis_index("s")
    pltpu.sync_copy(srcH.at[s], idx_s)
    pltpu.sync_copy(dstH.at[s], idx_d)

    def run(hw, out):
        _zero_acc(zrows, acc, s)
        plsc.subcore_barrier()
        _edge_loop(hw, acc, idx_s, idx_d, gbufs, gsems, ssems, CCH)
        plsc.subcore_barrier()
        _dump_acc(acc, out, s)
        plsc.subcore_barrier()

    @pl.when(c == 0)
    def _():
        run(hw0, out0)
        run(hw1, out1)

    @pl.when(c == 1)
    def _():
        run(hw2, out2)
        run(hw3, out3)


@functools.partial(
    pl.kernel,
    out_type=jax.ShapeDtypeStruct((N, D_OUT), jnp.float32),
    mesh=_mesh,
    compiler_params=_params,
    scratch_types=[
        pltpu.VMEM((CCH, KH), jnp.int32),
        pltpu.VMEM((CCH, KH), jnp.int32),
        [pltpu.VMEM((KH, DH2), jnp.float32)] * NB,
        pltpu.VMEM_SHARED((N, DH2), jnp.float32),
        pltpu.VMEM((RCH + RTAIL, DH2), jnp.float32),
        pltpu.VMEM((RCH + RTAIL, DH2), jnp.float32),
        pltpu.VMEM((2, DH2), jnp.float32),
        [pltpu.SemaphoreType.DMA] * NB,
        [pltpu.SemaphoreType.DMA] * NB,
    ],
)
def _agg_o(g0, g1, srcH, dstH, zrows, nrm32, b2h, out,
           idx_s, idx_d, gbufs, acc, tbuf, nbuf, b2v, gsems, ssems):
    c = lax.axis_index("c")
    s = lax.axis_index("s")
    pltpu.sync_copy(srcH.at[s], idx_s)
    pltpu.sync_copy(dstH.at[s], idx_d)
    pltpu.sync_copy(b2h, b2v)
    pltpu.sync_copy(nrm32.at[pl.ds(s * RCH, RCH)], nbuf.at[pl.ds(0, RCH)])

    @pl.when(s == NS - 1)
    def _():
        pltpu.sync_copy(nrm32.at[pl.ds(NS * RCH, RTAIL)],
                        nbuf.at[pl.ds(RCH, RTAIL)])

    def run(g, q):
        _zero_acc(zrows, acc, s)
        plsc.subcore_barrier()
        _edge_loop(g, acc, idx_s, idx_d, gbufs, gsems, ssems, CCH)
        plsc.subcore_barrier()
        pltpu.sync_copy(acc.at[pl.ds(s * RCH, RCH)], tbuf.at[pl.ds(0, RCH)])

        @pl.when(s == NS - 1)
        def _():
            pltpu.sync_copy(acc.at[pl.ds(NS * RCH, RTAIL)],
                            tbuf.at[pl.ds(RCH, RTAIL)])

        def scale_rows(lo, nrows):
            def body(r, carry):
                for v in range(DH2 // 16):
                    sl = pl.ds(v * 16, 16)
                    tbuf[r, sl] = (tbuf[r, sl] * nbuf[r, sl]
                                   + b2v[q, sl])
                return carry
            lax.fori_loop(lo, lo + nrows, body, 0)

        scale_rows(0, RCH)

        @pl.when(s == NS - 1)
        def _():
            scale_rows(RCH, RTAIL)

        pltpu.sync_copy(tbuf.at[pl.ds(0, RCH)],
                        out.at[pl.ds(s * RCH, RCH), pl.ds(q * DH2, DH2)])

        @pl.when(s == NS - 1)
        def _():
            pltpu.sync_copy(tbuf.at[pl.ds(RCH, RTAIL)],
                            out.at[pl.ds(NS * RCH, RTAIL),
                                   pl.ds(q * DH2, DH2)])

    @pl.when(c == 0)
    def _():
        run(g0, 0)

    @pl.when(c == 1)
    def _():
        run(g1, 1)


def _tc1_body(x_ref, deg_ref, w_ref, o0, o1, o2, o3):
    n_out = lax.rsqrt(jnp.maximum(deg_ref[0, :, 0:1], 1.0))
    y = jnp.dot(x_ref[...] * n_out, w_ref[...],
                preferred_element_type=jnp.float32)
    dq = D_H // 4
    for q, o in enumerate((o0, o1, o2, o3)):
        o[...] = y[:, q * dq:(q + 1) * dq]


def _tc1(x, deg16, w1):
    return pl.pallas_call(
        _tc1_body,
        grid=(GRID,),
        in_specs=[
            pl.BlockSpec((BM, D_IN), lambda i: (i, 0)),
            pl.BlockSpec((1, BM, 16), lambda i: (0, i, 0)),
            pl.BlockSpec((D_IN, D_H), lambda i: (0, 0)),
        ],
        out_specs=[pl.BlockSpec((BM, D_H // 4), lambda i: (i, 0))] * 4,
        out_shape=[jax.ShapeDtypeStruct((N, D_H // 4), jnp.float32)] * 4,
    )(x, deg16, w1)


def _tc2_body(a0, a1, a2, a3, deg_ref, b1_ref, w_ref, o0, o1, on):
    n_out = lax.rsqrt(jnp.maximum(deg_ref[0, :, 0:1], 1.0))
    n_in = lax.rsqrt(jnp.maximum(deg_ref[1, :, 0:1], 1.0))
    agg = jnp.concatenate([a0[...], a1[...], a2[...], a3[...]], axis=1)
    h = jax.nn.relu(agg * n_in + b1_ref[...]) * n_out
    y = jnp.dot(h, w_ref[...], preferred_element_type=jnp.float32)
    for q, o in enumerate((o0, o1)):
        o[...] = y[:, q * DH2:(q + 1) * DH2]
    on[...] = jnp.broadcast_to(n_in, (BM, DH2))


def _tc2(aggs, deg16, b1, w2):
    return pl.pallas_call(
        _tc2_body,
        grid=(GRID,),
        in_specs=[pl.BlockSpec((BM, D_H // 4), lambda i: (i, 0))] * 4 + [
            pl.BlockSpec((2, BM, 16), lambda i: (0, i, 0)),
            pl.BlockSpec((1, D_H), lambda i: (0, 0)),
            pl.BlockSpec((D_H, D_OUT), lambda i: (0, 0)),
        ],
        out_specs=[pl.BlockSpec((BM, DH2), lambda i: (i, 0))] * 2 + [
            pl.BlockSpec((BM, DH2), lambda i: (i, 0))],
        out_shape=[jax.ShapeDtypeStruct((N, DH2), jnp.float32)] * 2 + [
            jax.ShapeDtypeStruct((N, DH2), jnp.float32)],
    )(*aggs, deg16, b1, w2)


def kernel(in_feat, edge_index, W1, b1, W2, b2):
    e4d = edge_index.reshape(NC, NS, CCD, KD)
    srcH = edge_index[0].reshape(NS, CCH, KH)
    dstH = edge_index[1].reshape(NS, CCH, KH)
    z16 = jnp.zeros((RCH, 16), jnp.float32)
    zh = jnp.zeros((RCH, D_H // 4), jnp.float32)
    zo = jnp.zeros((RCH, DH2), jnp.float32)
    ones_rows = jnp.zeros((KD, 16), jnp.float32).at[:, 0].set(1.0)

    deg16 = _deg_kernel(e4d, z16, ones_rows)
    hw = _tc1(in_feat, deg16, W1)
    a1 = _agg_h(*hw, srcH, dstH, zh)
    g0, g1, nrm32 = _tc2(a1, deg16, b1.reshape(1, D_H), W2)
    return _agg_o(g0, g1, srcH, dstH, zo, nrm32, b2.reshape(2, DH2))


# speedup vs baseline: 1.2104x; 1.0871x over previous
import functools

import jax
import jax.numpy as jnp
from jax import lax
from jax.experimental import pallas as pl
from jax.experimental.pallas import tpu as pltpu
from jax.experimental.pallas import tpu_sc as plsc

N = 10000
E = 160000
D_IN = 256
D_H = 256
D_OUT = 64

NS = 16
NC = 2
NB = 5

KD = 80
CCD = (E // NS) // KD

KH = 80
CCH = (E // NS) // KH

DH2 = D_OUT // 2

RCH = 624
RTAIL = N - NS * RCH

BM = 2000
GRID = N // BM

_mesh = plsc.VectorSubcoreMesh(core_axis_name="c", subcore_axis_name="s")
_params = pltpu.CompilerParams(use_tc_tiling_on_sc=False)


def _zero_acc(zrows, acc, s):
    pltpu.sync_copy(zrows.at[pl.ds(0, RCH)], acc.at[pl.ds(s * RCH, RCH)])

    @pl.when(s == NS - 1)
    def _():
        pltpu.sync_copy(zrows.at[pl.ds(0, RTAIL)],
                        acc.at[pl.ds(NS * RCH, RTAIL)])


def _dump_acc(acc, out, s):
    pltpu.sync_copy(acc.at[pl.ds(s * RCH, RCH)], out.at[pl.ds(s * RCH, RCH)])

    @pl.when(s == NS - 1)
    def _():
        pltpu.sync_copy(acc.at[pl.ds(NS * RCH, RTAIL)],
                        out.at[pl.ds(NS * RCH, RTAIL)])


def _edge_loop(hw, acc, idx_s, idx_d, gbufs, gsems, ssems, cch):
    nb = len(gbufs)
    t_outer = cch // nb
    for b in range(nb - 1):
        pltpu.async_copy(hw.at[idx_s.at[b]], gbufs[b], gsems[b])

    def outer(t, carry):
        for b in range(nb):
            jj = t * nb + b
            bprev = (b - 1) % nb

            def wait_scatter(bp=bprev, j=jj):
                pltpu.make_async_copy(gbufs[bp], acc.at[idx_d.at[j - 1]],
                                      ssems[bp]).wait()

            def issue_gather(bp=bprev, j=jj):
                pltpu.async_copy(hw.at[idx_s.at[j + nb - 1]], gbufs[bp],
                                 gsems[bp])

            if b == 0:
                pl.when(t > 0)(wait_scatter)
                issue_gather()
            else:
                wait_scatter()
                pl.when(t < t_outer - 1)(issue_gather)
            pltpu.make_async_copy(hw.at[idx_s.at[jj]], gbufs[b],
                                  gsems[b]).wait()
            pltpu.async_copy(gbufs[b], acc.at[idx_d.at[jj]], ssems[b],
                             add=True)
        return carry

    lax.fori_loop(0, t_outer, outer, 0)
    bl = (cch - 1) % nb
    pltpu.make_async_copy(gbufs[bl], acc.at[idx_d.at[cch - 1]],
                          ssems[bl]).wait()


@functools.partial(
    pl.kernel,
    out_type=jax.ShapeDtypeStruct((NC, N, 16), jnp.float32),
    mesh=_mesh,
    compiler_params=_params,
    scratch_types=[
        pltpu.VMEM((CCD, KD), jnp.int32),
        pltpu.VMEM((KD, 16), jnp.float32),
        pltpu.VMEM_SHARED((N, 16), jnp.float32),
        [pltpu.SemaphoreType.DMA] * NB,
    ],
)
def _deg_kernel(e4, zrows, ones_rows, deg_out, idx, ones_v, deg_sp, dsems):
    c = lax.axis_index("c")
    s = lax.axis_index("s")
    _zero_acc(zrows, deg_sp, s)
    pltpu.sync_copy(e4.at[c, s], idx)
    pltpu.sync_copy(ones_rows, ones_v)
    plsc.subcore_barrier()

    def outer(t, carry):
        for b in range(NB):
            jj = t * NB + b

            def wait_prev(b=b, jj=jj):
                pltpu.make_async_copy(ones_v, deg_sp.at[idx.at[jj - NB]],
                                      dsems[b]).wait()

            pl.when(t > 0)(wait_prev)
            pltpu.async_copy(ones_v, deg_sp.at[idx.at[jj]], dsems[b],
                             add=True)
        return carry

    lax.fori_loop(0, CCD // NB, outer, 0)
    for b in range(NB):
        pltpu.make_async_copy(ones_v, deg_sp.at[idx.at[CCD - NB + b]],
                              dsems[b]).wait()
    plsc.subcore_barrier()
    _dump_acc(deg_sp, deg_out.at[c], s)


@functools.partial(
    pl.kernel,
    out_type=[jax.ShapeDtypeStruct((N, D_H // 4), jnp.float32)] * 4,
    mesh=_mesh,
    compiler_params=_params,
    scratch_types=[
        pltpu.VMEM((CCH, KH), jnp.int32),
        pltpu.VMEM((CCH, KH), jnp.int32),
        [pltpu.VMEM((KH, D_H // 4), jnp.float32)] * NB,
        pltpu.VMEM_SHARED((N, D_H // 4), jnp.float32),
        [pltpu.SemaphoreType.DMA] * NB,
        [pltpu.SemaphoreType.DMA] * NB,
    ],
)
def _agg_h(hw0, hw1, hw2, hw3, srcH, dstH, zrows,
           out0, out1, out2, out3, idx_s, idx_d, gbufs, acc, gsems, ssems):
    c = lax.axis_index("c")
    s = lax.axis_index("s")
    pltpu.sync_copy(srcH.at[s], idx_s)
    pltpu.sync_copy(dstH.at[s], idx_d)

    def run(hw, out):
        _zero_acc(zrows, acc, s)
        plsc.subcore_barrier()
        _edge_loop(hw, acc, idx_s, idx_d, gbufs, gsems, ssems, CCH)
        plsc.subcore_barrier()
        _dump_acc(acc, out, s)
        plsc.subcore_barrier()

    @pl.when(c == 0)
    def _():
        run(hw0, out0)
        run(hw1, out1)

    @pl.when(c == 1)
    def _():
        run(hw2, out2)
        run(hw3, out3)


@functools.partial(
    pl.kernel,
    out_type=jax.ShapeDtypeStruct((N, D_OUT), jnp.float32),
    mesh=_mesh,
    compiler_params=_params,
    scratch_types=[
        pltpu.VMEM((CCH, KH), jnp.int32),
        pltpu.VMEM((CCH, KH), jnp.int32),
        [pltpu.VMEM((KH, DH2), jnp.float32)] * NB,
        pltpu.VMEM_SHARED((N, DH2), jnp.float32),
        pltpu.VMEM((RCH + RTAIL, DH2), jnp.float32),
        pltpu.VMEM((RCH + RTAIL, DH2), jnp.float32),
        pltpu.VMEM((2, DH2), jnp.float32),
        [pltpu.SemaphoreType.DMA] * NB,
        [pltpu.SemaphoreType.DMA] * NB,
    ],
)
def _agg_o(g0, g1, srcH, dstH, zrows, nrm32, b2h, out,
           idx_s, idx_d, gbufs, acc, tbuf, nbuf, b2v, gsems, ssems):
    c = lax.axis_index("c")
    s = lax.axis_index("s")
    pltpu.sync_copy(srcH.at[s], idx_s)
    pltpu.sync_copy(dstH.at[s], idx_d)
    pltpu.sync_copy(b2h, b2v)
    pltpu.sync_copy(nrm32.at[pl.ds(s * RCH, RCH)], nbuf.at[pl.ds(0, RCH)])

    @pl.when(s == NS - 1)
    def _():
        pltpu.sync_copy(nrm32.at[pl.ds(NS * RCH, RTAIL)],
                        nbuf.at[pl.ds(RCH, RTAIL)])

    def run(g, q):
        _zero_acc(zrows, acc, s)
        plsc.subcore_barrier()
        _edge_loop(g, acc, idx_s, idx_d, gbufs, gsems, ssems, CCH)
        plsc.subcore_barrier()
        pltpu.sync_copy(acc.at[pl.ds(s * RCH, RCH)], tbuf.at[pl.ds(0, RCH)])

        @pl.when(s == NS - 1)
        def _():
            pltpu.sync_copy(acc.at[pl.ds(NS * RCH, RTAIL)],
                            tbuf.at[pl.ds(RCH, RTAIL)])

        def scale_rows(lo, nrows):
            def body(r, carry):
                for v in range(DH2 // 16):
                    sl = pl.ds(v * 16, 16)
                    tbuf[r, sl] = (tbuf[r, sl] * nbuf[r, sl]
                                   + b2v[q, sl])
                return carry
            lax.fori_loop(lo, lo + nrows, body, 0)

        scale_rows(0, RCH)

        @pl.when(s == NS - 1)
        def _():
            scale_rows(RCH, RTAIL)

        pltpu.sync_copy(tbuf.at[pl.ds(0, RCH)],
                        out.at[pl.ds(s * RCH, RCH), pl.ds(q * DH2, DH2)])

        @pl.when(s == NS - 1)
        def _():
            pltpu.sync_copy(tbuf.at[pl.ds(RCH, RTAIL)],
                            out.at[pl.ds(NS * RCH, RTAIL),
                                   pl.ds(q * DH2, DH2)])

    @pl.when(c == 0)
    def _():
        run(g0, 0)

    @pl.when(c == 1)
    def _():
        run(g1, 1)


def _tc1_body(x_ref, deg_ref, w_ref, o0, o1, o2, o3):
    n_out = lax.rsqrt(jnp.maximum(deg_ref[0, :, 0:1], 1.0))
    y = jnp.dot((x_ref[...] * n_out).astype(jnp.bfloat16),
                w_ref[...].astype(jnp.bfloat16),
                preferred_element_type=jnp.float32)
    dq = D_H // 4
    for q, o in enumerate((o0, o1, o2, o3)):
        o[...] = y[:, q * dq:(q + 1) * dq]


def _tc1(x, deg16, w1):
    return pl.pallas_call(
        _tc1_body,
        grid=(GRID,),
        in_specs=[
            pl.BlockSpec((BM, D_IN), lambda i: (i, 0)),
            pl.BlockSpec((1, BM, 16), lambda i: (0, i, 0)),
            pl.BlockSpec((D_IN, D_H), lambda i: (0, 0)),
        ],
        out_specs=[pl.BlockSpec((BM, D_H // 4), lambda i: (i, 0))] * 4,
        out_shape=[jax.ShapeDtypeStruct((N, D_H // 4), jnp.float32)] * 4,
    )(x, deg16, w1)


def _tc2_body(a0, a1, a2, a3, deg_ref, b1_ref, w_ref, o0, o1, on):
    n_out = lax.rsqrt(jnp.maximum(deg_ref[0, :, 0:1], 1.0))
    n_in = lax.rsqrt(jnp.maximum(deg_ref[1, :, 0:1], 1.0))
    agg = jnp.concatenate([a0[...], a1[...], a2[...], a3[...]], axis=1)
    h = jax.nn.relu(agg * n_in + b1_ref[...]) * n_out
    y = jnp.dot(h.astype(jnp.bfloat16), w_ref[...].astype(jnp.bfloat16),
                preferred_element_type=jnp.float32)
    for q, o in enumerate((o0, o1)):
        o[...] = y[:, q * DH2:(q + 1) * DH2]
    on[...] = jnp.broadcast_to(n_in, (BM, DH2))


def _tc2(aggs, deg16, b1, w2):
    return pl.pallas_call(
        _tc2_body,
        grid=(GRID,),
        in_specs=[pl.BlockSpec((BM, D_H // 4), lambda i: (i, 0))] * 4 + [
            pl.BlockSpec((2, BM, 16), lambda i: (0, i, 0)),
            pl.BlockSpec((1, D_H), lambda i: (0, 0)),
            pl.BlockSpec((D_H, D_OUT), lambda i: (0, 0)),
        ],
        out_specs=[pl.BlockSpec((BM, DH2), lambda i: (i, 0))] * 2 + [
            pl.BlockSpec((BM, DH2), lambda i: (i, 0))],
        out_shape=[jax.ShapeDtypeStruct((N, DH2), jnp.float32)] * 2 + [
            jax.ShapeDtypeStruct((N, DH2), jnp.float32)],
    )(*aggs, deg16, b1, w2)


def kernel(in_feat, edge_index, W1, b1, W2, b2):
    e4d = edge_index.reshape(NC, NS, CCD, KD)
    srcH = edge_index[0].reshape(NS, CCH, KH)
    dstH = edge_index[1].reshape(NS, CCH, KH)
    z16 = jnp.zeros((RCH, 16), jnp.float32)
    zh = jnp.zeros((RCH, D_H // 4), jnp.float32)
    zo = jnp.zeros((RCH, DH2), jnp.float32)
    ones_rows = jnp.zeros((KD, 16), jnp.float32).at[:, 0].set(1.0)

    deg16 = _deg_kernel(e4d, z16, ones_rows)
    hw = _tc1(in_feat, deg16, W1)
    a1 = _agg_h(*hw, srcH, dstH, zh)
    g0, g1, nrm32 = _tc2(a1, deg16, b1.reshape(1, D_H), W2)
    return _agg_o(g0, g1, srcH, dstH, zo, nrm32, b2.reshape(2, DH2))


# speedup vs baseline: 1.2116x; 1.0010x over previous
import functools

import jax
import jax.numpy as jnp
from jax import lax
from jax.experimental import pallas as pl
from jax.experimental.pallas import tpu as pltpu
from jax.experimental.pallas import tpu_sc as plsc

N = 10000
E = 160000
D_IN = 256
D_H = 256
D_OUT = 64

NS = 16
NC = 2
NB = 5

EPT = E // NS

KD = 80
CCD = (E // NS) // KD

KH = 80
CCH = (E // NS) // KH

DH2 = D_OUT // 2

RCH = 624
RTAIL = N - NS * RCH

BM = 2000
GRID = N // BM

_mesh = plsc.VectorSubcoreMesh(core_axis_name="c", subcore_axis_name="s")
_params = pltpu.CompilerParams(use_tc_tiling_on_sc=False)


def _zero_acc(zrows, acc, s):
    pltpu.sync_copy(zrows.at[pl.ds(0, RCH)], acc.at[pl.ds(s * RCH, RCH)])

    @pl.when(s == NS - 1)
    def _():
        pltpu.sync_copy(zrows.at[pl.ds(0, RTAIL)],
                        acc.at[pl.ds(NS * RCH, RTAIL)])


def _dump_acc(acc, out, s):
    pltpu.sync_copy(acc.at[pl.ds(s * RCH, RCH)], out.at[pl.ds(s * RCH, RCH)])

    @pl.when(s == NS - 1)
    def _():
        pltpu.sync_copy(acc.at[pl.ds(NS * RCH, RTAIL)],
                        out.at[pl.ds(NS * RCH, RTAIL)])


def _edge_loop(hw, acc, idx_s, idx_d, gbufs, gsems, ssems, cch, k):
    nb = len(gbufs)
    t_outer = cch // nb

    def sl(ref, j):
        return ref.at[pl.ds(j * k, k)]

    for b in range(nb - 1):
        pltpu.async_copy(hw.at[sl(idx_s, b)], gbufs[b], gsems[b])

    def outer(t, carry):
        for b in range(nb):
            jj = t * nb + b
            bprev = (b - 1) % nb

            def wait_scatter(bp=bprev, j=jj):
                pltpu.make_async_copy(gbufs[bp], acc.at[sl(idx_d, j - 1)],
                                      ssems[bp]).wait()

            def issue_gather(bp=bprev, j=jj):
                pltpu.async_copy(hw.at[sl(idx_s, j + nb - 1)], gbufs[bp],
                                 gsems[bp])

            if b == 0:
                pl.when(t > 0)(wait_scatter)
                issue_gather()
            else:
                wait_scatter()
                pl.when(t < t_outer - 1)(issue_gather)
            pltpu.make_async_copy(hw.at[sl(idx_s, jj)], gbufs[b],
                                  gsems[b]).wait()
            pltpu.async_copy(gbufs[b], acc.at[sl(idx_d, jj)], ssems[b],
                             add=True)
        return carry

    lax.fori_loop(0, t_outer, outer, 0)
    bl = (cch - 1) % nb
    pltpu.make_async_copy(gbufs[bl], acc.at[sl(idx_d, cch - 1)],
                          ssems[bl]).wait()


@functools.partial(
    pl.kernel,
    out_type=jax.ShapeDtypeStruct((NC, N, 16), jnp.float32),
    mesh=_mesh,
    compiler_params=_params,
    scratch_types=[
        pltpu.VMEM((EPT,), jnp.int32),
        pltpu.VMEM((KD, 16), jnp.float32),
        pltpu.VMEM_SHARED((N, 16), jnp.float32),
        [pltpu.SemaphoreType.DMA] * NB,
    ],
)
def _deg_kernel(edge, zrows, ones_rows, deg_out, idx, ones_v, deg_sp, dsems):
    c = lax.axis_index("c")
    s = lax.axis_index("s")
    _zero_acc(zrows, deg_sp, s)
    pltpu.sync_copy(edge.at[c, pl.ds(s * EPT, EPT)], idx)
    pltpu.sync_copy(ones_rows, ones_v)
    plsc.subcore_barrier()

    def outer(t, carry):
        for b in range(NB):
            jj = t * NB + b

            def wait_prev(b=b, jj=jj):
                pltpu.make_async_copy(
                    ones_v, deg_sp.at[idx.at[pl.ds((jj - NB) * KD, KD)]],
                    dsems[b]).wait()

            pl.when(t > 0)(wait_prev)
            pltpu.async_copy(ones_v, deg_sp.at[idx.at[pl.ds(jj * KD, KD)]],
                             dsems[b], add=True)
        return carry

    lax.fori_loop(0, CCD // NB, outer, 0)
    for b in range(NB):
        pltpu.make_async_copy(
            ones_v, deg_sp.at[idx.at[pl.ds((CCD - NB + b) * KD, KD)]],
            dsems[b]).wait()
    plsc.subcore_barrier()
    _dump_acc(deg_sp, deg_out.at[c], s)


@functools.partial(
    pl.kernel,
    out_type=[jax.ShapeDtypeStruct((N, D_H // 4), jnp.float32)] * 4,
    mesh=_mesh,
    compiler_params=_params,
    scratch_types=[
        pltpu.VMEM((EPT,), jnp.int32),
        pltpu.VMEM((EPT,), jnp.int32),
        [pltpu.VMEM((KH, D_H // 4), jnp.float32)] * NB,
        pltpu.VMEM_SHARED((N, D_H // 4), jnp.float32),
        [pltpu.SemaphoreType.DMA] * NB,
        [pltpu.SemaphoreType.DMA] * NB,
    ],
)
def _agg_h(hw0, hw1, hw2, hw3, edge, zrows,
           out0, out1, out2, out3, idx_s, idx_d, gbufs, acc, gsems, ssems):
    c = lax.axis_index("c")
    s = lax.axis_index("s")
    pltpu.sync_copy(edge.at[0, pl.ds(s * EPT, EPT)], idx_s)
    pltpu.sync_copy(edge.at[1, pl.ds(s * EPT, EPT)], idx_d)

    def run(hw, out):
        _zero_acc(zrows, acc, s)
        plsc.subcore_barrier()
        _edge_loop(hw, acc, idx_s, idx_d, gbufs, gsems, ssems, CCH, KH)
        plsc.subcore_barrier()
        _dump_acc(acc, out, s)
        plsc.subcore_barrier()

    @pl.when(c == 0)
    def _():
        run(hw0, out0)
        run(hw1, out1)

    @pl.when(c == 1)
    def _():
        run(hw2, out2)
        run(hw3, out3)


@functools.partial(
    pl.kernel,
    out_type=jax.ShapeDtypeStruct((N, D_OUT), jnp.float32),
    mesh=_mesh,
    compiler_params=_params,
    scratch_types=[
        pltpu.VMEM((EPT,), jnp.int32),
        pltpu.VMEM((EPT,), jnp.int32),
        [pltpu.VMEM((KH, DH2), jnp.float32)] * NB,
        pltpu.VMEM_SHARED((N, DH2), jnp.float32),
        pltpu.VMEM((RCH + RTAIL, DH2), jnp.float32),
        pltpu.VMEM((RCH + RTAIL, DH2), jnp.float32),
        pltpu.VMEM((2, DH2), jnp.float32),
        [pltpu.SemaphoreType.DMA] * NB,
        [pltpu.SemaphoreType.DMA] * NB,
    ],
)
def _agg_o(g0, g1, edge, zrows, nrm32, b2h, out,
           idx_s, idx_d, gbufs, acc, tbuf, nbuf, b2v, gsems, ssems):
    c = lax.axis_index("c")
    s = lax.axis_index("s")
    pltpu.sync_copy(edge.at[0, pl.ds(s * EPT, EPT)], idx_s)
    pltpu.sync_copy(edge.at[1, pl.ds(s * EPT, EPT)], idx_d)
    pltpu.sync_copy(b2h, b2v)
    pltpu.sync_copy(nrm32.at[pl.ds(s * RCH, RCH)], nbuf.at[pl.ds(0, RCH)])

    @pl.when(s == NS - 1)
    def _():
        pltpu.sync_copy(nrm32.at[pl.ds(NS * RCH, RTAIL)],
                        nbuf.at[pl.ds(RCH, RTAIL)])

    def run(g, q):
        _zero_acc(zrows, acc, s)
        plsc.subcore_barrier()
        _edge_loop(g, acc, idx_s, idx_d, gbufs, gsems, ssems, CCH, KH)
        plsc.subcore_barrier()
        pltpu.sync_copy(acc.at[pl.ds(s * RCH, RCH)], tbuf.at[pl.ds(0, RCH)])

        @pl.when(s == NS - 1)
        def _():
            pltpu.sync_copy(acc.at[pl.ds(NS * RCH, RTAIL)],
                            tbuf.at[pl.ds(RCH, RTAIL)])

        def scale_rows(lo, nrows):
            def body(r, carry):
                for v in range(DH2 // 16):
                    sl = pl.ds(v * 16, 16)
                    tbuf[r, sl] = (tbuf[r, sl] * nbuf[r, sl]
                                   + b2v[q, sl])
                return carry
            lax.fori_loop(lo, lo + nrows, body, 0)

        scale_rows(0, RCH)

        @pl.when(s == NS - 1)
        def _():
            scale_rows(RCH, RTAIL)

        pltpu.sync_copy(tbuf.at[pl.ds(0, RCH)],
                        out.at[pl.ds(s * RCH, RCH), pl.ds(q * DH2, DH2)])

        @pl.when(s == NS - 1)
        def _():
            pltpu.sync_copy(tbuf.at[pl.ds(RCH, RTAIL)],
                            out.at[pl.ds(NS * RCH, RTAIL),
                                   pl.ds(q * DH2, DH2)])

    @pl.when(c == 0)
    def _():
        run(g0, 0)

    @pl.when(c == 1)
    def _():
        run(g1, 1)


def _tc1_body(x_ref, deg_ref, w_ref, o0, o1, o2, o3):
    n_out = lax.rsqrt(jnp.maximum(deg_ref[0, :, 0:1], 1.0))
    y = jnp.dot((x_ref[...] * n_out).astype(jnp.bfloat16),
                w_ref[...].astype(jnp.bfloat16),
                preferred_element_type=jnp.float32)
    dq = D_H // 4
    for q, o in enumerate((o0, o1, o2, o3)):
        o[...] = y[:, q * dq:(q + 1) * dq]


def _tc1(x, deg16, w1):
    return pl.pallas_call(
        _tc1_body,
        grid=(GRID,),
        in_specs=[
            pl.BlockSpec((BM, D_IN), lambda i: (i, 0)),
            pl.BlockSpec((1, BM, 16), lambda i: (0, i, 0)),
            pl.BlockSpec((D_IN, D_H), lambda i: (0, 0)),
        ],
        out_specs=[pl.BlockSpec((BM, D_H // 4), lambda i: (i, 0))] * 4,
        out_shape=[jax.ShapeDtypeStruct((N, D_H // 4), jnp.float32)] * 4,
    )(x, deg16, w1)


def _tc2_body(a0, a1, a2, a3, deg_ref, b1_ref, w_ref, o0, o1, on):
    n_out = lax.rsqrt(jnp.maximum(deg_ref[0, :, 0:1], 1.0))
    n_in = lax.rsqrt(jnp.maximum(deg_ref[1, :, 0:1], 1.0))
    agg = jnp.concatenate([a0[...], a1[...], a2[...], a3[...]], axis=1)
    h = jax.nn.relu(agg * n_in + b1_ref[...]) * n_out
    y = jnp.dot(h.astype(jnp.bfloat16), w_ref[...].astype(jnp.bfloat16),
                preferred_element_type=jnp.float32)
    for q, o in enumerate((o0, o1)):
        o[...] = y[:, q * DH2:(q + 1) * DH2]
    on[...] = jnp.broadcast_to(n_in, (BM, DH2))


def _tc2(aggs, deg16, b1, w2):
    return pl.pallas_call(
        _tc2_body,
        grid=(GRID,),
        in_specs=[pl.BlockSpec((BM, D_H // 4), lambda i: (i, 0))] * 4 + [
            pl.BlockSpec((2, BM, 16), lambda i: (0, i, 0)),
            pl.BlockSpec((1, D_H), lambda i: (0, 0)),
            pl.BlockSpec((D_H, D_OUT), lambda i: (0, 0)),
        ],
        out_specs=[pl.BlockSpec((BM, DH2), lambda i: (i, 0))] * 2 + [
            pl.BlockSpec((BM, DH2), lambda i: (i, 0))],
        out_shape=[jax.ShapeDtypeStruct((N, DH2), jnp.float32)] * 2 + [
            jax.ShapeDtypeStruct((N, DH2), jnp.float32)],
    )(*aggs, deg16, b1, w2)


def kernel(in_feat, edge_index, W1, b1, W2, b2):
    z16 = jnp.zeros((RCH, 16), jnp.float32)
    zh = jnp.zeros((RCH, D_H // 4), jnp.float32)
    zo = jnp.zeros((RCH, DH2), jnp.float32)
    ones_rows = jnp.zeros((KD, 16), jnp.float32).at[:, 0].set(1.0)

    deg16 = _deg_kernel(edge_index, z16, ones_rows)
    hw = _tc1(in_feat, deg16, W1)
    a1 = _agg_h(*hw, edge_index, zh)
    g0, g1, nrm32 = _tc2(a1, deg16, b1.reshape(1, D_H), W2)
    return _agg_o(g0, g1, edge_index, zo, nrm32, b2.reshape(2, DH2))
